# Initial kernel scaffold; baseline (speedup 1.0000x reference)
import jax
import jax.numpy as jnp
from jax.experimental import pallas as pl


def kernel(x, Wg, W1, b1, W2, b2, Ws1, bs1, Ws2, bs2, gamma, beta):
    raise NotImplementedError("write your pallas kernel here")


# speedup vs baseline: 1.8612x; 1.8612x over previous
import functools

import jax
import jax.numpy as jnp
from jax import lax
from jax.experimental import pallas as pl
from jax.experimental.pallas import tpu as pltpu

T = 4096
NC = 2048
NI = 4096
NS = 1024
E = 8
BM = 1024
BNI = 512


def _gate_body(x_ref, wg_ref, c_ref):
    s = lax.dot_general(x_ref[...], wg_ref[...],
                        (((1,), (1,)), ((), ())),
                        preferred_element_type=jnp.float32)
    s = jax.nn.softmax(s, axis=-1)
    iota = lax.broadcasted_iota(jnp.int32, s.shape, 1)
    m1 = jnp.max(s, axis=-1, keepdims=True)
    e1 = jnp.min(jnp.where(s == m1, iota, E), axis=-1, keepdims=True)
    sel1 = iota == e1
    s2 = jnp.where(sel1, -jnp.inf, s)
    m2 = jnp.max(s2, axis=-1, keepdims=True)
    e2 = jnp.min(jnp.where(s2 == m2, iota, E), axis=-1, keepdims=True)
    sel2 = iota == e2
    c_ref[...] = jnp.where(sel1, m1, 0.0) + jnp.where(sel2, m2, 0.0)


def _experts_body(x_ref, w1_ref, b1_ref, w2_ref, b2_ref, c_ref, y_ref):
    j = pl.program_id(1)
    k = pl.program_id(2)

    @pl.when(jnp.logical_and(j == 0, k == 0))
    def _():
        y_ref[...] = jnp.zeros_like(y_ref)

    onehot = (lax.broadcasted_iota(jnp.int32, (BM, E), 1) == j)
    c_sel = jnp.sum(jnp.where(onehot, c_ref[...], 0.0), axis=-1,
                    keepdims=True)

    h = lax.dot_general(x_ref[...], w1_ref[0],
                        (((1,), (1,)), ((), ())),
                        preferred_element_type=jnp.float32)
    h = h + b1_ref[0]
    h = 0.5 * h * (1.0 + lax.erf(h * 0.7071067811865476))
    part = lax.dot_general(h, w2_ref[0],
                           (((1,), (1,)), ((), ())),
                           preferred_element_type=jnp.float32)
    acc = part * c_sel

    @pl.when(k == 0)
    def _():
        y_ref[...] += b2_ref[0] * c_sel

    y_ref[...] += acc


def _shared_body(x_ref, ws1_ref, bs1_ref, ws2_ref, bs2_ref, z_ref):
    k = pl.program_id(1)

    @pl.when(k == 0)
    def _():
        z_ref[...] = jnp.broadcast_to(bs2_ref[...], z_ref.shape)

    h = lax.dot_general(x_ref[...], ws1_ref[...],
                        (((1,), (1,)), ((), ())),
                        preferred_element_type=jnp.float32) + bs1_ref[...]
    z_ref[...] += lax.dot_general(h, ws2_ref[...],
                                  (((1,), (1,)), ((), ())),
                                  preferred_element_type=jnp.float32)


def _ln_body(y_ref, z_ref, g_ref, b_ref, o_ref):
    v = (y_ref[...] + z_ref[...]) * 0.5
    mean = jnp.mean(v, axis=-1, keepdims=True)
    var = jnp.mean(jnp.square(v - mean), axis=-1, keepdims=True)
    o_ref[...] = (v - mean) * lax.rsqrt(var + 1e-5) * g_ref[...] + b_ref[...]


def kernel(x, Wg, W1, b1, W2, b2, Ws1, bs1, Ws2, bs2, gamma, beta):
    bsz, nf, nc = x.shape
    xf = x.reshape(T, NC)

    combine = pl.pallas_call(
        _gate_body,
        grid=(T // BM,),
        in_specs=[
            pl.BlockSpec((BM, NC), lambda i: (i, 0)),
            pl.BlockSpec((E, NC), lambda i: (0, 0)),
        ],
        out_specs=pl.BlockSpec((BM, E), lambda i: (i, 0)),
        out_shape=jax.ShapeDtypeStruct((T, E), jnp.float32),
    )(xf, Wg)

    y = pl.pallas_call(
        _experts_body,
        grid=(T // BM, E, NI // BNI),
        in_specs=[
            pl.BlockSpec((BM, NC), lambda i, j, k: (i, 0)),
            pl.BlockSpec((1, BNI, NC), lambda i, j, k: (j, k, 0)),
            pl.BlockSpec((1, 1, BNI), lambda i, j, k: (j * (NI // BNI) + k, 0, 0)),
            pl.BlockSpec((1, NS, BNI), lambda i, j, k: (j, 0, k)),
            pl.BlockSpec((1, 1, NS), lambda i, j, k: (j, 0, 0)),
            pl.BlockSpec((BM, E), lambda i, j, k: (i, 0)),
        ],
        out_specs=pl.BlockSpec((BM, NS), lambda i, j, k: (i, 0)),
        out_shape=jax.ShapeDtypeStruct((T, NS), jnp.float32),
        compiler_params=pltpu.CompilerParams(
            dimension_semantics=("parallel", "arbitrary", "arbitrary")),
    )(xf, W1, b1.reshape(E * (NI // BNI), 1, BNI), W2, b2.reshape(E, 1, NS),
      combine)

    z = pl.pallas_call(
        _shared_body,
        grid=(T // BM, NI // BNI),
        in_specs=[
            pl.BlockSpec((BM, NC), lambda i, k: (i, 0)),
            pl.BlockSpec((BNI, NC), lambda i, k: (k, 0)),
            pl.BlockSpec((1, BNI), lambda i, k: (0, k)),
            pl.BlockSpec((NS, BNI), lambda i, k: (0, k)),
            pl.BlockSpec((1, NS), lambda i, k: (0, 0)),
        ],
        out_specs=pl.BlockSpec((BM, NS), lambda i, k: (i, 0)),
        out_shape=jax.ShapeDtypeStruct((T, NS), jnp.float32),
        compiler_params=pltpu.CompilerParams(
            dimension_semantics=("parallel", "arbitrary")),
    )(xf, Ws1, bs1.reshape(1, NI), Ws2, bs2.reshape(1, NS))

    out = pl.pallas_call(
        _ln_body,
        grid=(T // BM,),
        in_specs=[
            pl.BlockSpec((BM, NS), lambda i: (i, 0)),
            pl.BlockSpec((BM, NS), lambda i: (i, 0)),
            pl.BlockSpec((1, NS), lambda i: (0, 0)),
            pl.BlockSpec((1, NS), lambda i: (0, 0)),
        ],
        out_specs=pl.BlockSpec((BM, NS), lambda i: (i, 0)),
        out_shape=jax.ShapeDtypeStruct((T, NS), jnp.float32),
    )(y, z, gamma.reshape(1, NS), beta.reshape(1, NS))

    return out.reshape(bsz, nf, NS)
